# Initial kernel scaffold; baseline (speedup 1.0000x reference)
import jax
import jax.numpy as jnp
from jax.experimental import pallas as pl


def kernel(points_ref, points):
    raise NotImplementedError("write your pallas kernel here")


# speedup vs baseline: 20.8593x; 20.8593x over previous
import functools

import jax
import jax.numpy as jnp
from jax.experimental import pallas as pl
from jax.experimental.pallas import tpu as pltpu

_B = 4
_N = 4096
_K = 17
_R = 256
_IMAX = jnp.iinfo(jnp.int32).max


def _tile_body(qr, kr, qp, kp, out_ref, keys_ref, km_ref, t_ref):
    qr = qr[0]
    kr = kr[0]
    qp = qp[0]
    kp = kp[0]

    dn = (((0,), (0,)), ((), ()))
    qk = jax.lax.dot_general(qr, kr, dn, precision=jax.lax.Precision.DEFAULT,
                             preferred_element_type=jnp.float32)
    q2 = jnp.sum(qr * qr, axis=0)
    r2 = jnp.sum(kr * kr, axis=0)
    d2 = q2[:, None] + r2[None, :] - 2.0 * qk

    ci = jax.lax.broadcasted_iota(jnp.int32, (_R, _N), 1)
    sb = jax.lax.bitcast_convert_type(d2, jnp.int32)
    mono = sb ^ jax.lax.shift_right_logical(
        jax.lax.shift_right_arithmetic(sb, 31), 1)
    keys = (mono & ~0xFFF) | ci
    keys_ref[...] = keys
    km_ref[...] = keys

    dd = jnp.zeros((_R, _N), jnp.float32)
    ee = jnp.zeros((_R, _N), jnp.float32)
    for c in range(3):
        dr = qr[c][:, None] - kr[c][None, :]
        dd = dd + dr * dr
        dp = qp[c][:, None] - kp[c][None, :]
        ee = ee + dp * dp
    t_ref[...] = jnp.abs(jnp.sqrt(dd) - jnp.sqrt(ee))

    def _extract(i, m0):
        km = km_ref[...]
        m = jnp.min(km, axis=1, keepdims=True)
        km_ref[...] = jnp.where(km == m, _IMAX, km)
        return jnp.where(i == 0, m, m0)

    m0 = jax.lax.fori_loop(0, _K, _extract, jnp.zeros((_R, 1), jnp.int32),
                           unroll=True)

    sel = jnp.logical_and(km_ref[...] == _IMAX, keys_ref[...] != m0)
    step = jnp.sum(jnp.where(sel, t_ref[...], 0.0)) * (1.0 / (_B * _N * (_K - 1)))

    b = pl.program_id(0)
    r = pl.program_id(1)

    @pl.when(jnp.logical_and(b == 0, r == 0))
    def _init():
        out_ref[0, 0] = 0.0

    out_ref[0, 0] += step


def kernel(points_ref, points):
    ref_t = jnp.transpose(points_ref, (0, 2, 1))
    pts_t = jnp.transpose(points, (0, 2, 1))

    grid = (_B, _N // _R)
    out = pl.pallas_call(
        _tile_body,
        grid=grid,
        in_specs=[
            pl.BlockSpec((1, 3, _R), lambda b, r: (b, 0, r)),
            pl.BlockSpec((1, 3, _N), lambda b, r: (b, 0, 0)),
            pl.BlockSpec((1, 3, _R), lambda b, r: (b, 0, r)),
            pl.BlockSpec((1, 3, _N), lambda b, r: (b, 0, 0)),
        ],
        out_specs=pl.BlockSpec((1, 1), lambda b, r: (0, 0),
                               memory_space=pltpu.SMEM),
        out_shape=jax.ShapeDtypeStruct((1, 1), jnp.float32),
        scratch_shapes=[
            pltpu.VMEM((_R, _N), jnp.int32),
            pltpu.VMEM((_R, _N), jnp.int32),
            pltpu.VMEM((_R, _N), jnp.float32),
        ],
    )(ref_t, ref_t, pts_t, pts_t)
    return jnp.reshape(out, ())


# speedup vs baseline: 21.3532x; 1.0237x over previous
import functools

import jax
import jax.numpy as jnp
from jax.experimental import pallas as pl
from jax.experimental.pallas import tpu as pltpu

_B = 4
_N = 4096
_K = 17
_R = 256
_IMAX = jnp.iinfo(jnp.int32).max


def _tile_body(qr, kr, qp, kp, out_ref, keys_ref):
    qr = qr[0]
    kr = kr[0]
    qp = qp[0]
    kp = kp[0]

    dn = (((0,), (0,)), ((), ()))
    qk = jax.lax.dot_general(qr, kr, dn, precision=jax.lax.Precision.DEFAULT,
                             preferred_element_type=jnp.float32)
    q2 = jnp.sum(qr * qr, axis=0)
    r2 = jnp.sum(kr * kr, axis=0)
    d2 = q2[:, None] + r2[None, :] - 2.0 * qk

    ci = jax.lax.broadcasted_iota(jnp.int32, (_R, _N), 1)
    sb = jax.lax.bitcast_convert_type(d2, jnp.int32)
    mono = sb ^ jax.lax.shift_right_logical(
        jax.lax.shift_right_arithmetic(sb, 31), 1)
    keys_ref[...] = (mono & ~0xFFF) | ci

    keys = keys_ref[...]
    m0 = jnp.min(keys, axis=1, keepdims=True)

    def _next(i, m):
        return jnp.min(jnp.where(keys > m, keys, _IMAX), axis=1,
                       keepdims=True)

    t = jax.lax.fori_loop(0, _K - 1, _next, m0, unroll=True)

    dd = jnp.zeros((_R, _N), jnp.float32)
    ee = jnp.zeros((_R, _N), jnp.float32)
    for c in range(3):
        dr = qr[c][:, None] - kr[c][None, :]
        dd = dd + dr * dr
        dp = qp[c][:, None] - kp[c][None, :]
        ee = ee + dp * dp

    sel = jnp.logical_and(keys <= t, keys != m0)
    term = jnp.abs(jnp.sqrt(dd) - jnp.sqrt(ee))
    step = jnp.sum(jnp.where(sel, term, 0.0)) * (1.0 / (_B * _N * (_K - 1)))

    b = pl.program_id(0)
    r = pl.program_id(1)

    @pl.when(jnp.logical_and(b == 0, r == 0))
    def _init():
        out_ref[0, 0] = 0.0

    out_ref[0, 0] += step


def kernel(points_ref, points):
    ref_t = jnp.transpose(points_ref, (0, 2, 1))
    pts_t = jnp.transpose(points, (0, 2, 1))

    grid = (_B, _N // _R)
    out = pl.pallas_call(
        _tile_body,
        grid=grid,
        in_specs=[
            pl.BlockSpec((1, 3, _R), lambda b, r: (b, 0, r)),
            pl.BlockSpec((1, 3, _N), lambda b, r: (b, 0, 0)),
            pl.BlockSpec((1, 3, _R), lambda b, r: (b, 0, r)),
            pl.BlockSpec((1, 3, _N), lambda b, r: (b, 0, 0)),
        ],
        out_specs=pl.BlockSpec((1, 1), lambda b, r: (0, 0),
                               memory_space=pltpu.SMEM),
        out_shape=jax.ShapeDtypeStruct((1, 1), jnp.float32),
        scratch_shapes=[
            pltpu.VMEM((_R, _N), jnp.int32),
        ],
    )(ref_t, ref_t, pts_t, pts_t)
    return jnp.reshape(out, ())


# speedup vs baseline: 26.3665x; 1.2348x over previous
import functools

import jax
import jax.numpy as jnp
from jax.experimental import pallas as pl
from jax.experimental.pallas import tpu as pltpu

_B = 4
_N = 4096
_K = 17
_R = 256


def _tile_body(qr, kr, qp, kp, out_ref, keys_ref):
    qr = qr[0]
    kr = kr[0]
    qp = qp[0]
    kp = kp[0]

    dn = (((0,), (0,)), ((), ()))
    qk = jax.lax.dot_general(qr, kr, dn, precision=jax.lax.Precision.DEFAULT,
                             preferred_element_type=jnp.float32)
    q2 = jnp.sum(qr * qr, axis=0)
    r2 = jnp.sum(kr * kr, axis=0)
    d2 = q2[:, None] + r2[None, :] - 2.0 * qk

    ci = jax.lax.broadcasted_iota(jnp.int32, (_R, _N), 1)
    sb = jax.lax.bitcast_convert_type(d2, jnp.int32)
    cif = ci ^ (jax.lax.shift_right_arithmetic(sb, 31) & 0xFFF)
    keys_ref[...] = jax.lax.bitcast_convert_type((sb & ~0xFFF) | cif,
                                                 jnp.float32)

    keys = keys_ref[...]
    m0 = jnp.min(keys, axis=1, keepdims=True)

    def _next(i, m):
        return jnp.min(jnp.where(keys > m, keys, jnp.inf), axis=1,
                       keepdims=True)

    t = jax.lax.fori_loop(0, _K - 1, _next, m0, unroll=True)

    qkh = jax.lax.dot_general(qr, kr, dn, precision=jax.lax.Precision.HIGHEST,
                              preferred_element_type=jnp.float32)
    dd = jnp.maximum(q2[:, None] + r2[None, :] - 2.0 * qkh, 0.0)
    qph = jax.lax.dot_general(qp, kp, dn, precision=jax.lax.Precision.HIGHEST,
                              preferred_element_type=jnp.float32)
    p2 = jnp.sum(qp * qp, axis=0)
    s2 = jnp.sum(kp * kp, axis=0)
    ee = jnp.maximum(p2[:, None] + s2[None, :] - 2.0 * qph, 0.0)

    sel = jnp.logical_and(keys <= t, keys != m0)
    term = jnp.abs(jnp.sqrt(dd) - jnp.sqrt(ee))
    step = jnp.sum(jnp.where(sel, term, 0.0)) * (1.0 / (_B * _N * (_K - 1)))

    b = pl.program_id(0)
    r = pl.program_id(1)

    @pl.when(jnp.logical_and(b == 0, r == 0))
    def _init():
        out_ref[0, 0] = 0.0

    out_ref[0, 0] += step


def kernel(points_ref, points):
    ref_t = jnp.transpose(points_ref, (0, 2, 1))
    pts_t = jnp.transpose(points, (0, 2, 1))

    grid = (_B, _N // _R)
    out = pl.pallas_call(
        _tile_body,
        grid=grid,
        in_specs=[
            pl.BlockSpec((1, 3, _R), lambda b, r: (b, 0, r)),
            pl.BlockSpec((1, 3, _N), lambda b, r: (b, 0, 0)),
            pl.BlockSpec((1, 3, _R), lambda b, r: (b, 0, r)),
            pl.BlockSpec((1, 3, _N), lambda b, r: (b, 0, 0)),
        ],
        out_specs=pl.BlockSpec((1, 1), lambda b, r: (0, 0),
                               memory_space=pltpu.SMEM),
        out_shape=jax.ShapeDtypeStruct((1, 1), jnp.float32),
        scratch_shapes=[
            pltpu.VMEM((_R, _N), jnp.float32),
        ],
    )(ref_t, ref_t, pts_t, pts_t)
    return jnp.reshape(out, ())


# speedup vs baseline: 40.0845x; 1.5203x over previous
import functools

import jax
import jax.numpy as jnp
from jax import lax
from jax.experimental import pallas as pl
from jax.experimental.pallas import tpu as pltpu
from jax.experimental.pallas import tpu_sc as plsc

_B = 4
_N = 4096
_K = 17
_R = 256
_NC = 2
_NS = 16
_NW = _NC * _NS
_NPW = (_B * _N) // _NW


def _tc_body(qr, kr, out_ref, keys_ref):
    qr = qr[0]
    kr = kr[0]

    dn = (((0,), (0,)), ((), ()))
    qk = jax.lax.dot_general(qr, kr, dn, precision=jax.lax.Precision.DEFAULT,
                             preferred_element_type=jnp.float32)
    q2 = jnp.sum(qr * qr, axis=0)
    r2 = jnp.sum(kr * kr, axis=0)
    d2 = q2[:, None] + r2[None, :] - 2.0 * qk

    ci = jax.lax.broadcasted_iota(jnp.int32, (_R, _N), 1)
    sb = jax.lax.bitcast_convert_type(d2, jnp.int32)
    cif = ci ^ (jax.lax.shift_right_arithmetic(sb, 31) & 0xFFF)
    keys_ref[...] = jax.lax.bitcast_convert_type((sb & ~0xFFF) | cif,
                                                 jnp.float32)

    keys = keys_ref[...]
    m0 = jnp.min(keys, axis=1, keepdims=True)

    kiT = jax.lax.broadcasted_iota(jnp.int32, (_K - 1, _R), 0)
    colsT = jnp.zeros((_K - 1, _R), jnp.int32)

    def _next(i, carry):
        m, colsT = carry
        m = jnp.min(jnp.where(keys > m, keys, jnp.inf), axis=1,
                    keepdims=True)
        mb = jax.lax.bitcast_convert_type(m, jnp.int32)
        col = (mb & 0xFFF) ^ (jax.lax.shift_right_arithmetic(mb, 31) & 0xFFF)
        colsT = jnp.where(kiT == i, jnp.reshape(col, (1, _R)), colsT)
        return m, colsT

    _, colsT = jax.lax.fori_loop(0, _K - 1, _next, (m0, colsT), unroll=True)
    out_ref[0] = colsT


def _nsqrt(s):
    i = jax.lax.bitcast_convert_type(s, jnp.int32)
    x = jax.lax.bitcast_convert_type(
        jax.lax.shift_right_arithmetic(i, 1) + 0x1FBD1DF6, jnp.float32)
    x = 0.5 * (x + s / x)
    x = 0.5 * (x + s / x)
    return x


def _sc_body(rxh, ryh, rzh, pxh, pyh, pzh, cols_hbm, out_hbm,
             rx, ry, rz, px, py, pz, cv, st, acc):
    wid = lax.axis_index("s") * _NC + lax.axis_index("c")
    b = wid // (_NW // _B)
    n0 = (wid % (_NW // _B)) * _NPW

    pltpu.sync_copy(rxh.at[pl.ds(b * _N, _N)], rx)
    pltpu.sync_copy(ryh.at[pl.ds(b * _N, _N)], ry)
    pltpu.sync_copy(rzh.at[pl.ds(b * _N, _N)], rz)
    pltpu.sync_copy(pxh.at[pl.ds(b * _N, _N)], px)
    pltpu.sync_copy(pyh.at[pl.ds(b * _N, _N)], py)
    pltpu.sync_copy(pzh.at[pl.ds(b * _N, _N)], pz)
    pltpu.sync_copy(cols_hbm.at[b, :, pl.ds(n0, _NPW)], cv)

    acc[...] = jnp.zeros((16,), jnp.float32)

    def _group(g, _):
        base = g * 16
        qx = rx[pl.ds(n0 + base, 16)]
        qy = ry[pl.ds(n0 + base, 16)]
        qz = rz[pl.ds(n0 + base, 16)]
        sx = px[pl.ds(n0 + base, 16)]
        sy = py[pl.ds(n0 + base, 16)]
        sz = pz[pl.ds(n0 + base, 16)]
        for k in range(_K - 1):
            idx = cv[k, pl.ds(base, 16)]
            gx = plsc.load_gather(rx, [idx])
            gy = plsc.load_gather(ry, [idx])
            gz = plsc.load_gather(rz, [idx])
            dx = gx - qx
            dy = gy - qy
            dz = gz - qz
            dd = dx * dx + dy * dy + dz * dz
            hx = plsc.load_gather(px, [idx])
            hy = plsc.load_gather(py, [idx])
            hz = plsc.load_gather(pz, [idx])
            ex = hx - sx
            ey = hy - sy
            ez = hz - sz
            ee = ex * ex + ey * ey + ez * ez
            acc[...] += jnp.abs(_nsqrt(dd) - _nsqrt(ee))
        return _

    lax.fori_loop(0, _NPW // 16, _group, 0)
    st[...] = acc[...]
    pltpu.sync_copy(st, out_hbm.at[wid])


@functools.cache
def _sc_kernel():
    return pl.kernel(
        _sc_body,
        out_type=jax.ShapeDtypeStruct((_NW, 16), jnp.float32),
        mesh=plsc.VectorSubcoreMesh(core_axis_name="c",
                                    subcore_axis_name="s"),
        compiler_params=pltpu.CompilerParams(needs_layout_passes=False),
        scratch_types=[
            pltpu.VMEM((_N,), jnp.float32),
            pltpu.VMEM((_N,), jnp.float32),
            pltpu.VMEM((_N,), jnp.float32),
            pltpu.VMEM((_N,), jnp.float32),
            pltpu.VMEM((_N,), jnp.float32),
            pltpu.VMEM((_N,), jnp.float32),
            pltpu.VMEM((_K - 1, _NPW), jnp.int32),
            pltpu.VMEM((16,), jnp.float32),
            pltpu.VMEM((16,), jnp.float32),
        ],
    )


def kernel(points_ref, points):
    ref_t = jnp.transpose(points_ref, (0, 2, 1))
    pts_t = jnp.transpose(points, (0, 2, 1))

    cols = pl.pallas_call(
        _tc_body,
        grid=(_B, _N // _R),
        in_specs=[
            pl.BlockSpec((1, 3, _R), lambda b, r: (b, 0, r)),
            pl.BlockSpec((1, 3, _N), lambda b, r: (b, 0, 0)),
        ],
        out_specs=pl.BlockSpec((1, _K - 1, _R), lambda b, r: (b, 0, r)),
        out_shape=jax.ShapeDtypeStruct((_B, _K - 1, _N), jnp.int32),
        scratch_shapes=[
            pltpu.VMEM((_R, _N), jnp.float32),
        ],
    )(ref_t, ref_t)

    partials = _sc_kernel()(
        ref_t[:, 0].reshape(-1), ref_t[:, 1].reshape(-1),
        ref_t[:, 2].reshape(-1), pts_t[:, 0].reshape(-1),
        pts_t[:, 1].reshape(-1), pts_t[:, 2].reshape(-1), cols)
    return jnp.sum(partials) * (1.0 / (_B * _N * (_K - 1)))
